# Initial kernel scaffold; baseline (speedup 1.0000x reference)
import jax
import jax.numpy as jnp
from jax.experimental import pallas as pl


def kernel(edge_index, feat, W_feat, b_feat, W_layers, b_layers, W_cls, b_cls):
    raise NotImplementedError("write your pallas kernel here")


# speedup vs baseline: 4.0380x; 4.0380x over previous
import functools

import jax
import jax.numpy as jnp
from jax import lax
from jax.experimental import pallas as pl
from jax.experimental.pallas import tpu as pltpu
from jax.experimental.pallas import tpu_sc as plsc

_CH = 128
_QCH = 16


@functools.lru_cache(maxsize=None)
def _make_edge_agg(N, D, NC, NS, NCHUNK, NPAD):
    NZ = NPAD // (NS * _CH)
    ROWS_T = NPAD // NS
    QCH = _QCH
    NQ = NCHUNK // QCH
    assert NCHUNK % QCH == 0 and QCH % 2 == 0
    mesh = plsc.VectorSubcoreMesh(core_axis_name="c", subcore_axis_name="s",
                                  num_cores=NC, num_subcores=NS)

    @functools.partial(
        pl.kernel,
        out_type=jax.ShapeDtypeStruct((NC, NPAD, D), jnp.float32),
        mesh=mesh,
        scratch_types=[
            pltpu.VMEM((QCH, _CH), jnp.int32),
            pltpu.VMEM((QCH, _CH), jnp.int32),
            pltpu.VMEM((_CH, D), jnp.float32),
            pltpu.VMEM((_CH, D), jnp.float32),
            pltpu.VMEM_SHARED((NPAD, D), jnp.float32),
            pltpu.SemaphoreType.DMA,
            pltpu.SemaphoreType.DMA,
        ],
    )
    def edge_agg(h_hbm, src_hbm, dst_hbm, zeros_hbm, out_hbm,
                 src_v, dst_v, buf0, buf1, agg_sh, sem0, sem1):
        c = lax.axis_index("c")
        s = lax.axis_index("s")
        wid = s * NC + c
        pltpu.sync_copy(zeros_hbm, buf0)

        def zero_step(t, carry):
            pltpu.sync_copy(buf0, agg_sh.at[pl.ds((s * NZ + t) * _CH, _CH)])
            return carry

        lax.fori_loop(0, NZ, zero_step, 0)
        plsc.subcore_barrier()

        def quarter(q, carry):
            pltpu.sync_copy(src_hbm.at[wid, pl.ds(q * QCH, QCH)], src_v)
            pltpu.sync_copy(dst_hbm.at[wid, pl.ds(q * QCH, QCH)], dst_v)
            pltpu.async_copy(h_hbm.at[src_v.at[0]], buf0, sem0)
            pltpu.async_copy(h_hbm.at[src_v.at[1]], buf1, sem1)

            def step(t, carry2):
                j = 2 * t
                pltpu.make_async_copy(h_hbm.at[src_v.at[j]], buf0,
                                      sem0).wait()
                pltpu.sync_copy(buf0, agg_sh.at[dst_v.at[j]], add=True)

                @pl.when(j + 2 < QCH)
                def _():
                    pltpu.async_copy(h_hbm.at[src_v.at[j + 2]], buf0, sem0)

                pltpu.make_async_copy(h_hbm.at[src_v.at[j + 1]], buf1,
                                      sem1).wait()
                pltpu.sync_copy(buf1, agg_sh.at[dst_v.at[j + 1]], add=True)

                @pl.when(j + 3 < QCH)
                def _():
                    pltpu.async_copy(h_hbm.at[src_v.at[j + 3]], buf1, sem1)

                return carry2

            lax.fori_loop(0, QCH // 2, step, 0)
            return carry

        lax.fori_loop(0, NQ, quarter, 0)
        plsc.subcore_barrier()
        pltpu.sync_copy(agg_sh.at[pl.ds(s * ROWS_T, ROWS_T)],
                        out_hbm.at[c, pl.ds(s * ROWS_T, ROWS_T)])

    return edge_agg


def _linear_body(x_ref, w_ref, b_ref, o_ref):
    o_ref[...] = (jnp.dot(x_ref[...], w_ref[...],
                          preferred_element_type=jnp.float32) + b_ref[...])


def _linear(x, w, b, block_m):
    M, K = x.shape
    Dh = w.shape[1]
    return pl.pallas_call(
        _linear_body,
        grid=(M // block_m,),
        in_specs=[
            pl.BlockSpec((block_m, K), lambda i: (i, 0)),
            pl.BlockSpec((K, Dh), lambda i: (0, 0)),
            pl.BlockSpec((1, Dh), lambda i: (0, 0)),
        ],
        out_specs=pl.BlockSpec((block_m, Dh), lambda i: (i, 0)),
        out_shape=jax.ShapeDtypeStruct((M, Dh), jnp.float32),
    )(x, w, b)


def _layer_body(a_ref, w_ref, b_ref, o_ref):
    ssum = a_ref[0] + a_ref[1]
    o_ref[...] = jnp.maximum(
        jnp.dot(ssum, w_ref[...], preferred_element_type=jnp.float32)
        + b_ref[...], 0.0)


def _layer(agg, w, b, n_rows, block_m):
    NCpart, _, K = agg.shape
    Dh = w.shape[1]
    return pl.pallas_call(
        _layer_body,
        grid=(n_rows // block_m,),
        in_specs=[
            pl.BlockSpec((NCpart, block_m, K), lambda i: (0, i, 0)),
            pl.BlockSpec((K, Dh), lambda i: (0, 0)),
            pl.BlockSpec((1, Dh), lambda i: (0, 0)),
        ],
        out_specs=pl.BlockSpec((block_m, Dh), lambda i: (i, 0)),
        out_shape=jax.ShapeDtypeStruct((n_rows, Dh), jnp.float32),
    )(agg, w, b)


def _final_body(a_ref, w_ref, b_ref, wc_ref, bc_ref, o_ref, *,
                nblocks, n_nodes):
    i = pl.program_id(0)
    ssum = a_ref[0] + a_ref[1]
    hblk = jnp.maximum(
        jnp.dot(ssum, w_ref[...], preferred_element_type=jnp.float32)
        + b_ref[...], 0.0)
    part = jnp.sum(hblk * wc_ref[...]) / n_nodes
    prev = jnp.where(i == 0, bc_ref[0, 0], o_ref[0, 0])
    o_ref[0, 0] = prev + part


def _final(agg, w, b, wc_row, bc, n_rows, block_m):
    NCpart, _, K = agg.shape
    Dh = w.shape[1]
    nblocks = n_rows // block_m
    return pl.pallas_call(
        functools.partial(_final_body, nblocks=nblocks, n_nodes=n_rows),
        grid=(nblocks,),
        in_specs=[
            pl.BlockSpec((NCpart, block_m, K), lambda i: (0, i, 0)),
            pl.BlockSpec((K, Dh), lambda i: (0, 0)),
            pl.BlockSpec((1, Dh), lambda i: (0, 0)),
            pl.BlockSpec((1, Dh), lambda i: (0, 0)),
            pl.BlockSpec(memory_space=pltpu.SMEM),
        ],
        out_specs=pl.BlockSpec(memory_space=pltpu.SMEM),
        out_shape=jax.ShapeDtypeStruct((1, 1), jnp.float32),
    )(agg, w, b, wc_row, bc)


def kernel(edge_index, feat, W_feat, b_feat, W_layers, b_layers, W_cls, b_cls):
    N, D_feat = feat.shape
    D_hid = W_feat.shape[1]
    L = W_layers.shape[0]
    E = edge_index.shape[1]

    info = plsc.get_sparse_core_info()
    NC, NS = info.num_cores, info.num_subcores
    NW = NC * NS

    NCHUNK = -(-(-(-E // (NW * _CH))) // _QCH) * _QCH
    per_w = NCHUNK * _CH
    E_pad = per_w * NW
    src = edge_index[0].astype(jnp.int32)
    dst = edge_index[1].astype(jnp.int32)
    src_p = jnp.concatenate([src, jnp.zeros((E_pad - E,), jnp.int32)])
    dst_p = jnp.concatenate([dst, jnp.full((E_pad - E,), N, jnp.int32)])
    src_mat = src_p.reshape(NW, NCHUNK, _CH)
    dst_mat = dst_p.reshape(NW, NCHUNK, _CH)
    NPAD = -(-(N + 1) // (NS * _CH)) * (NS * _CH)
    zeros_tile = jnp.zeros((_CH, D_hid), jnp.float32)

    edge_agg = _make_edge_agg(N, D_hid, NC, NS, NCHUNK, NPAD)

    block_m = 1000
    h = _linear(feat, W_feat, b_feat.reshape(1, D_hid), block_m)
    for i in range(L - 1):
        agg = edge_agg(h, src_mat, dst_mat, zeros_tile)
        h = _layer(agg, W_layers[i], b_layers[i].reshape(1, D_hid), N,
                   block_m)
    agg = edge_agg(h, src_mat, dst_mat, zeros_tile)
    p = _final(agg, W_layers[L - 1], b_layers[L - 1].reshape(1, D_hid),
               W_cls.reshape(1, D_hid), b_cls.reshape(1, 1), N, block_m)
    return p.reshape(1)


# speedup vs baseline: 4.5092x; 1.1167x over previous
import functools

import jax
import jax.numpy as jnp
from jax import lax
from jax.experimental import pallas as pl
from jax.experimental.pallas import tpu as pltpu
from jax.experimental.pallas import tpu_sc as plsc

_CH = 512
_QF = 5


@functools.lru_cache(maxsize=None)
def _make_edge_agg(N, DH, NC, NS, NCHUNK, NPAD):
    NZ = NPAD // (NS * 128)
    ROWS_T = NPAD // NS
    QCH = NCHUNK // _QF
    assert NCHUNK % _QF == 0 and QCH % 8 == 0
    mesh = plsc.VectorSubcoreMesh(core_axis_name="c", subcore_axis_name="s",
                                  num_cores=NC, num_subcores=NS)

    @functools.partial(
        pl.kernel,
        out_type=jax.ShapeDtypeStruct((NC, NPAD, DH), jnp.float32),
        mesh=mesh,
        compiler_params=pltpu.CompilerParams(use_tc_tiling_on_sc=False),
        scratch_types=[
            pltpu.VMEM((QCH * _CH,), jnp.int32),
            pltpu.VMEM((QCH * _CH,), jnp.int32),
            pltpu.VMEM((_CH, DH), jnp.float32),
            pltpu.VMEM((_CH, DH), jnp.float32),
            pltpu.VMEM_SHARED((NPAD, DH), jnp.float32),
            pltpu.SemaphoreType.DMA,
            pltpu.SemaphoreType.DMA,
        ],
    )
    def edge_agg(h_hbm, src_hbm, dst_hbm, zeros_hbm, out_hbm,
                 src_v, dst_v, buf0, buf1, agg_sh, sem0, sem1):
        c = lax.axis_index("c")
        s = lax.axis_index("s")
        pltpu.sync_copy(zeros_hbm, buf0.at[pl.ds(0, 128)])

        def zero_step(t, carry):
            pltpu.sync_copy(buf0.at[pl.ds(0, 128)],
                            agg_sh.at[pl.ds((s * NZ + t) * 128, 128)])
            return carry

        lax.fori_loop(0, NZ, zero_step, 0)
        plsc.subcore_barrier()

        def stage(q, carry):
            pltpu.sync_copy(src_hbm.at[c, s, pl.ds(q * QCH * _CH, QCH * _CH)],
                            src_v)
            pltpu.sync_copy(dst_hbm.at[s, pl.ds(q * QCH * _CH, QCH * _CH)],
                            dst_v)
            pltpu.async_copy(h_hbm.at[src_v.at[pl.ds(0, _CH)]], buf0, sem0)
            pltpu.async_copy(h_hbm.at[src_v.at[pl.ds(_CH, _CH)]], buf1, sem1)

            def step(t, carry2):
                j = 2 * t
                pltpu.make_async_copy(
                    h_hbm.at[src_v.at[pl.ds(j * _CH, _CH)]], buf0,
                    sem0).wait()
                pltpu.sync_copy(buf0,
                                agg_sh.at[dst_v.at[pl.ds(j * _CH, _CH)]],
                                add=True)

                @pl.when(j + 2 < QCH)
                def _():
                    pltpu.async_copy(
                        h_hbm.at[src_v.at[pl.ds((j + 2) * _CH, _CH)]],
                        buf0, sem0)

                pltpu.make_async_copy(
                    h_hbm.at[src_v.at[pl.ds((j + 1) * _CH, _CH)]], buf1,
                    sem1).wait()
                pltpu.sync_copy(buf1,
                                agg_sh.at[dst_v.at[pl.ds((j + 1) * _CH, _CH)]],
                                add=True)

                @pl.when(j + 3 < QCH)
                def _():
                    pltpu.async_copy(
                        h_hbm.at[src_v.at[pl.ds((j + 3) * _CH, _CH)]],
                        buf1, sem1)

                return carry2

            lax.fori_loop(0, QCH // 2, step, 0)
            return carry

        lax.fori_loop(0, _QF, stage, 0)
        plsc.subcore_barrier()
        pltpu.sync_copy(agg_sh.at[pl.ds(s * ROWS_T, ROWS_T)],
                        out_hbm.at[c, pl.ds(s * ROWS_T, ROWS_T)])

    return edge_agg


def _linear_body(x_ref, w_ref, b_ref, o_ref):
    o_ref[...] = (jnp.dot(x_ref[...], w_ref[0],
                          preferred_element_type=jnp.float32) + b_ref[0])


def _linear_split(x, w2, b2, n_rows, block_m):
    K = x.shape[1]
    DH = w2.shape[2]
    nblk = n_rows // block_m
    return pl.pallas_call(
        _linear_body,
        grid=(nblk, 2),
        in_specs=[
            pl.BlockSpec((block_m, K), lambda i, h: (i, 0)),
            pl.BlockSpec((1, K, DH), lambda i, h: (h, 0, 0)),
            pl.BlockSpec((1, 1, DH), lambda i, h: (h, 0, 0)),
        ],
        out_specs=pl.BlockSpec((block_m, DH), lambda i, h, _n=nblk:
                               (h * _n + i, 0)),
        out_shape=jax.ShapeDtypeStruct((2 * n_rows, DH), jnp.float32),
    )(x, w2, b2)


def _layer_body(a_ref, w_ref, b_ref, o_ref, *, dh):
    x = (jnp.dot(a_ref[0], w_ref[0, :dh, :],
                 preferred_element_type=jnp.float32)
         + jnp.dot(a_ref[1], w_ref[0, dh:, :],
                   preferred_element_type=jnp.float32))
    o_ref[...] = jnp.maximum(x + b_ref[0], 0.0)


def _layer_split(agg, w2, b2, n_rows, block_m):
    NCpart, _, DH = agg.shape
    nblk = n_rows // block_m
    return pl.pallas_call(
        functools.partial(_layer_body, dh=DH),
        grid=(nblk, 2),
        in_specs=[
            pl.BlockSpec((NCpart, block_m, DH), lambda i, h: (0, i, 0)),
            pl.BlockSpec((1, 2 * DH, DH), lambda i, h: (h, 0, 0)),
            pl.BlockSpec((1, 1, DH), lambda i, h: (h, 0, 0)),
        ],
        out_specs=pl.BlockSpec((block_m, DH), lambda i, h, _n=nblk:
                               (h * _n + i, 0)),
        out_shape=jax.ShapeDtypeStruct((2 * n_rows, DH), jnp.float32),
    )(agg, w2, b2)


def _final_body(a_ref, w_ref, b_ref, wc_ref, bc_ref, o_ref, *, dh, n_nodes):
    i = pl.program_id(0)
    x = (jnp.dot(a_ref[0], w_ref[:dh, :],
                 preferred_element_type=jnp.float32)
         + jnp.dot(a_ref[1], w_ref[dh:, :],
                   preferred_element_type=jnp.float32))
    hblk = jnp.maximum(x + b_ref[...], 0.0)
    part = jnp.sum(hblk * wc_ref[...]) / n_nodes
    prev = jnp.where(i == 0, bc_ref[0, 0], o_ref[0, 0])
    o_ref[0, 0] = prev + part


def _final(agg, w, b, wc_row, bc, n_rows, block_m):
    NCpart, _, DH = agg.shape
    D = w.shape[1]
    return pl.pallas_call(
        functools.partial(_final_body, dh=DH, n_nodes=n_rows),
        grid=(n_rows // block_m,),
        in_specs=[
            pl.BlockSpec((NCpart, block_m, DH), lambda i: (0, i, 0)),
            pl.BlockSpec((2 * DH, D), lambda i: (0, 0)),
            pl.BlockSpec((1, D), lambda i: (0, 0)),
            pl.BlockSpec((1, D), lambda i: (0, 0)),
            pl.BlockSpec(memory_space=pltpu.SMEM),
        ],
        out_specs=pl.BlockSpec(memory_space=pltpu.SMEM),
        out_shape=jax.ShapeDtypeStruct((1, 1), jnp.float32),
    )(agg, w, b, wc_row, bc)


def kernel(edge_index, feat, W_feat, b_feat, W_layers, b_layers, W_cls, b_cls):
    N = feat.shape[0]
    D_hid = W_feat.shape[1]
    DH = D_hid // 2
    L = W_layers.shape[0]
    E = edge_index.shape[1]

    info = plsc.get_sparse_core_info()
    NC, NS = info.num_cores, info.num_subcores
    NCHUNK = -(-(-(-E // (NS * _CH))) // (8 * _QF)) * (8 * _QF)
    E_pad = NCHUNK * _CH * NS
    src = edge_index[0].astype(jnp.int32)
    dst = edge_index[1].astype(jnp.int32)
    src_p = jnp.concatenate([src, jnp.zeros((E_pad - E,), jnp.int32)])
    dst_p = jnp.concatenate([dst, jnp.full((E_pad - E,), N, jnp.int32)])
    src_mat = jnp.stack([src_p, src_p + N]).reshape(NC, NS, NCHUNK * _CH)
    dst_mat = dst_p.reshape(NS, NCHUNK * _CH)
    NPAD = -(-(N + 1) // (NS * 128)) * (NS * 128)
    zeros_tile = jnp.zeros((128, DH), jnp.float32)

    edge_agg = _make_edge_agg(N, DH, NC, NS, NCHUNK, NPAD)

    def _colsplit(w):
        return w.reshape(w.shape[0], 2, DH).transpose(1, 0, 2)

    def _bsplit(b):
        return b.reshape(2, 1, DH)

    block_m = 1000
    h2 = _linear_split(feat, _colsplit(W_feat), _bsplit(b_feat), N, block_m)
    for i in range(L - 1):
        agg = edge_agg(h2, src_mat, dst_mat, zeros_tile)
        h2 = _layer_split(agg, _colsplit(W_layers[i]), _bsplit(b_layers[i]),
                          N, block_m)
    agg = edge_agg(h2, src_mat, dst_mat, zeros_tile)
    p = _final(agg, W_layers[L - 1], b_layers[L - 1].reshape(1, D_hid),
               W_cls.reshape(1, D_hid), b_cls.reshape(1, 1), N, block_m)
    return p.reshape(1)


# speedup vs baseline: 4.5535x; 1.0098x over previous
import functools

import jax
import jax.numpy as jnp
from jax import lax
from jax.experimental import pallas as pl
from jax.experimental.pallas import tpu as pltpu
from jax.experimental.pallas import tpu_sc as plsc

_CH = 256
_QF = 5
_NB = 4


@functools.lru_cache(maxsize=None)
def _make_edge_agg(N, DH, NC, NS, NCHUNK, NPAD):
    NZ = NPAD // (NS * 128)
    ROWS_T = NPAD // NS
    QCH = NCHUNK // _QF
    assert NCHUNK % _QF == 0 and QCH % 8 == 0 and QCH % _NB == 0
    mesh = plsc.VectorSubcoreMesh(core_axis_name="c", subcore_axis_name="s",
                                  num_cores=NC, num_subcores=NS)

    @functools.partial(
        pl.kernel,
        out_type=jax.ShapeDtypeStruct((NC, NPAD, DH), jnp.float32),
        mesh=mesh,
        compiler_params=pltpu.CompilerParams(use_tc_tiling_on_sc=False),
        scratch_types=[
            pltpu.VMEM((QCH * _CH,), jnp.int32),
            pltpu.VMEM((QCH * _CH,), jnp.int32),
            [pltpu.VMEM((_CH, DH), jnp.float32) for _ in range(_NB)],
            [pltpu.SemaphoreType.DMA for _ in range(_NB)],
            [pltpu.SemaphoreType.DMA for _ in range(_NB)],
            pltpu.VMEM_SHARED((NPAD, DH), jnp.float32),
        ],
    )
    def edge_agg(h_hbm, src_hbm, dst_hbm, zeros_hbm, out_hbm,
                 src_v, dst_v, bufs, gsem, ssem, agg_sh):
        c = lax.axis_index("c")
        s = lax.axis_index("s")

        def gather(j, b):
            pltpu.async_copy(h_hbm.at[src_v.at[pl.ds(j * _CH, _CH)]],
                             bufs[b], gsem[b])

        def gather_wait(j, b):
            pltpu.make_async_copy(h_hbm.at[src_v.at[pl.ds(j * _CH, _CH)]],
                                  bufs[b], gsem[b]).wait()

        def scatter(j, b):
            pltpu.async_copy(bufs[b],
                             agg_sh.at[dst_v.at[pl.ds(j * _CH, _CH)]],
                             ssem[b], add=True)

        def scatter_wait(j, b):
            pltpu.make_async_copy(bufs[b],
                                  agg_sh.at[dst_v.at[pl.ds(j * _CH, _CH)]],
                                  ssem[b]).wait()

        pltpu.sync_copy(zeros_hbm, bufs[0].at[pl.ds(0, 128)])

        def zero_step(t, carry):
            pltpu.sync_copy(bufs[0].at[pl.ds(0, 128)],
                            agg_sh.at[pl.ds((s * NZ + t) * 128, 128)])
            return carry

        lax.fori_loop(0, NZ, zero_step, 0)
        plsc.subcore_barrier()

        NT = QCH // _NB

        def stage(q, carry):
            pltpu.sync_copy(src_hbm.at[c, s, pl.ds(q * QCH * _CH, QCH * _CH)],
                            src_v)
            pltpu.sync_copy(dst_hbm.at[s, pl.ds(q * QCH * _CH, QCH * _CH)],
                            dst_v)
            for b in range(_NB - 1):
                gather(b, b)

            def step(t, carry2):
                for u in range(_NB):
                    j = _NB * t + u
                    b = u
                    gather_wait(j, b)
                    scatter(j, b)
                    bn = (u + _NB - 1) % _NB
                    if u == 0:
                        @pl.when(t > 0)
                        def _():
                            scatter_wait(j - 1, bn)
                        gather(j + _NB - 1, bn)
                    else:
                        scatter_wait(j - 1, bn)

                        @pl.when(t < NT - 1)
                        def _():
                            gather(j + _NB - 1, bn)
                return carry2

            lax.fori_loop(0, NT, step, 0)
            scatter_wait(QCH - 1, _NB - 1)
            return carry

        lax.fori_loop(0, _QF, stage, 0)
        plsc.subcore_barrier()
        pltpu.sync_copy(agg_sh.at[pl.ds(s * ROWS_T, ROWS_T)],
                        out_hbm.at[c, pl.ds(s * ROWS_T, ROWS_T)])

    return edge_agg


def _linear_body(x_ref, w_ref, b_ref, o_ref):
    o_ref[...] = (jnp.dot(x_ref[...], w_ref[0],
                          preferred_element_type=jnp.float32) + b_ref[0])


def _linear_split(x, w2, b2, n_rows, block_m):
    K = x.shape[1]
    DH = w2.shape[2]
    nblk = n_rows // block_m
    return pl.pallas_call(
        _linear_body,
        grid=(nblk, 2),
        in_specs=[
            pl.BlockSpec((block_m, K), lambda i, h: (i, 0)),
            pl.BlockSpec((1, K, DH), lambda i, h: (h, 0, 0)),
            pl.BlockSpec((1, 1, DH), lambda i, h: (h, 0, 0)),
        ],
        out_specs=pl.BlockSpec((block_m, DH), lambda i, h, _n=nblk:
                               (h * _n + i, 0)),
        out_shape=jax.ShapeDtypeStruct((2 * n_rows, DH), jnp.float32),
    )(x, w2, b2)


def _layer_body(a_ref, w_ref, b_ref, o_ref, *, dh):
    x = (jnp.dot(a_ref[0], w_ref[0, :dh, :],
                 preferred_element_type=jnp.float32)
         + jnp.dot(a_ref[1], w_ref[0, dh:, :],
                   preferred_element_type=jnp.float32))
    o_ref[...] = jnp.maximum(x + b_ref[0], 0.0)


def _layer_split(agg, w2, b2, n_rows, block_m):
    NCpart, _, DH = agg.shape
    nblk = n_rows // block_m
    return pl.pallas_call(
        functools.partial(_layer_body, dh=DH),
        grid=(nblk, 2),
        in_specs=[
            pl.BlockSpec((NCpart, block_m, DH), lambda i, h: (0, i, 0)),
            pl.BlockSpec((1, 2 * DH, DH), lambda i, h: (h, 0, 0)),
            pl.BlockSpec((1, 1, DH), lambda i, h: (h, 0, 0)),
        ],
        out_specs=pl.BlockSpec((block_m, DH), lambda i, h, _n=nblk:
                               (h * _n + i, 0)),
        out_shape=jax.ShapeDtypeStruct((2 * n_rows, DH), jnp.float32),
    )(agg, w2, b2)


def _final_body(a_ref, w_ref, b_ref, wc_ref, bc_ref, o_ref, *, dh, n_nodes):
    i = pl.program_id(0)
    x = (jnp.dot(a_ref[0], w_ref[:dh, :],
                 preferred_element_type=jnp.float32)
         + jnp.dot(a_ref[1], w_ref[dh:, :],
                   preferred_element_type=jnp.float32))
    hblk = jnp.maximum(x + b_ref[...], 0.0)
    part = jnp.sum(hblk * wc_ref[...]) / n_nodes
    prev = jnp.where(i == 0, bc_ref[0, 0], o_ref[0, 0])
    o_ref[0, 0] = prev + part


def _final(agg, w, b, wc_row, bc, n_rows, block_m):
    NCpart, _, DH = agg.shape
    D = w.shape[1]
    return pl.pallas_call(
        functools.partial(_final_body, dh=DH, n_nodes=n_rows),
        grid=(n_rows // block_m,),
        in_specs=[
            pl.BlockSpec((NCpart, block_m, DH), lambda i: (0, i, 0)),
            pl.BlockSpec((2 * DH, D), lambda i: (0, 0)),
            pl.BlockSpec((1, D), lambda i: (0, 0)),
            pl.BlockSpec((1, D), lambda i: (0, 0)),
            pl.BlockSpec(memory_space=pltpu.SMEM),
        ],
        out_specs=pl.BlockSpec(memory_space=pltpu.SMEM),
        out_shape=jax.ShapeDtypeStruct((1, 1), jnp.float32),
    )(agg, w, b, wc_row, bc)


def kernel(edge_index, feat, W_feat, b_feat, W_layers, b_layers, W_cls, b_cls):
    N = feat.shape[0]
    D_hid = W_feat.shape[1]
    DH = D_hid // 2
    L = W_layers.shape[0]
    E = edge_index.shape[1]

    info = plsc.get_sparse_core_info()
    NC, NS = info.num_cores, info.num_subcores
    NCHUNK = -(-(-(-E // (NS * _CH))) // (8 * _QF)) * (8 * _QF)
    E_pad = NCHUNK * _CH * NS
    src = edge_index[0].astype(jnp.int32)
    dst = edge_index[1].astype(jnp.int32)
    src_p = jnp.concatenate([src, jnp.zeros((E_pad - E,), jnp.int32)])
    dst_p = jnp.concatenate([dst, jnp.full((E_pad - E,), N, jnp.int32)])
    src_mat = jnp.stack([src_p, src_p + N]).reshape(NC, NS, NCHUNK * _CH)
    dst_mat = dst_p.reshape(NS, NCHUNK * _CH)
    NPAD = -(-(N + 1) // (NS * 128)) * (NS * 128)
    zeros_tile = jnp.zeros((128, DH), jnp.float32)

    edge_agg = _make_edge_agg(N, DH, NC, NS, NCHUNK, NPAD)

    def _colsplit(w):
        return w.reshape(w.shape[0], 2, DH).transpose(1, 0, 2)

    def _bsplit(b):
        return b.reshape(2, 1, DH)

    block_m = 1000
    h2 = _linear_split(feat, _colsplit(W_feat), _bsplit(b_feat), N, block_m)
    for i in range(L - 1):
        agg = edge_agg(h2, src_mat, dst_mat, zeros_tile)
        h2 = _layer_split(agg, _colsplit(W_layers[i]), _bsplit(b_layers[i]),
                          N, block_m)
    agg = edge_agg(h2, src_mat, dst_mat, zeros_tile)
    p = _final(agg, W_layers[L - 1], b_layers[L - 1].reshape(1, D_hid),
               W_cls.reshape(1, D_hid), b_cls.reshape(1, 1), N, block_m)
    return p.reshape(1)


# speedup vs baseline: 8.0009x; 1.7571x over previous
import functools

import jax
import jax.numpy as jnp
from jax import lax
from jax.experimental import pallas as pl
from jax.experimental.pallas import tpu as pltpu
from jax.experimental.pallas import tpu_sc as plsc

_CH = 256
_QF = 5


@functools.lru_cache(maxsize=None)
def _make_edge_agg(N, DH, NC, NS, NCHUNK, NPAD):
    NZ = NPAD // (NS * 128)
    ROWS_T = NPAD // NS
    QCH = NCHUNK // _QF
    HROWS = N // NS
    assert NCHUNK % _QF == 0 and QCH % 8 == 0
    mesh = plsc.VectorSubcoreMesh(core_axis_name="c", subcore_axis_name="s",
                                  num_cores=NC, num_subcores=NS)

    @functools.partial(
        pl.kernel,
        out_type=jax.ShapeDtypeStruct((NC, NPAD, DH), jnp.float32),
        mesh=mesh,
        compiler_params=pltpu.CompilerParams(use_tc_tiling_on_sc=False),
        scratch_types=[
            pltpu.VMEM((QCH * _CH,), jnp.int32),
            pltpu.VMEM((QCH * _CH,), jnp.int32),
            pltpu.VMEM((_CH, DH), jnp.float32),
            pltpu.VMEM((_CH, DH), jnp.float32),
            pltpu.VMEM_SHARED((N, DH), jnp.float32),
            pltpu.VMEM_SHARED((NPAD, DH), jnp.float32),
            pltpu.SemaphoreType.DMA,
            pltpu.SemaphoreType.DMA,
        ],
    )
    def edge_agg(h_hbm, src_hbm, dst_hbm, zeros_hbm, out_hbm,
                 src_v, dst_v, buf0, buf1, h_sh, agg_sh, sem0, sem1):
        c = lax.axis_index("c")
        s = lax.axis_index("s")
        pltpu.sync_copy(h_hbm.at[pl.ds(c * N + s * HROWS, HROWS)],
                        h_sh.at[pl.ds(s * HROWS, HROWS)])
        pltpu.sync_copy(zeros_hbm, buf0.at[pl.ds(0, 128)])

        def zero_step(t, carry):
            pltpu.sync_copy(buf0.at[pl.ds(0, 128)],
                            agg_sh.at[pl.ds((s * NZ + t) * 128, 128)])
            return carry

        lax.fori_loop(0, NZ, zero_step, 0)
        plsc.subcore_barrier()

        def stage(q, carry):
            pltpu.sync_copy(src_hbm.at[s, pl.ds(q * QCH * _CH, QCH * _CH)],
                            src_v)
            pltpu.sync_copy(dst_hbm.at[s, pl.ds(q * QCH * _CH, QCH * _CH)],
                            dst_v)
            pltpu.async_copy(h_sh.at[src_v.at[pl.ds(0, _CH)]], buf0, sem0)
            pltpu.async_copy(h_sh.at[src_v.at[pl.ds(_CH, _CH)]], buf1, sem1)

            def step(t, carry2):
                j = 2 * t
                pltpu.make_async_copy(
                    h_sh.at[src_v.at[pl.ds(j * _CH, _CH)]], buf0,
                    sem0).wait()
                pltpu.sync_copy(buf0,
                                agg_sh.at[dst_v.at[pl.ds(j * _CH, _CH)]],
                                add=True)

                @pl.when(j + 2 < QCH)
                def _():
                    pltpu.async_copy(
                        h_sh.at[src_v.at[pl.ds((j + 2) * _CH, _CH)]],
                        buf0, sem0)

                pltpu.make_async_copy(
                    h_sh.at[src_v.at[pl.ds((j + 1) * _CH, _CH)]], buf1,
                    sem1).wait()
                pltpu.sync_copy(buf1,
                                agg_sh.at[dst_v.at[pl.ds((j + 1) * _CH, _CH)]],
                                add=True)

                @pl.when(j + 3 < QCH)
                def _():
                    pltpu.async_copy(
                        h_sh.at[src_v.at[pl.ds((j + 3) * _CH, _CH)]],
                        buf1, sem1)

                return carry2

            lax.fori_loop(0, QCH // 2, step, 0)
            return carry

        lax.fori_loop(0, _QF, stage, 0)
        plsc.subcore_barrier()
        pltpu.sync_copy(agg_sh.at[pl.ds(s * ROWS_T, ROWS_T)],
                        out_hbm.at[c, pl.ds(s * ROWS_T, ROWS_T)])

    return edge_agg


def _linear_body(x_ref, w_ref, b_ref, o_ref):
    o_ref[...] = (jnp.dot(x_ref[...], w_ref[0],
                          preferred_element_type=jnp.float32) + b_ref[0])


def _linear_split(x, w2, b2, n_rows, block_m):
    K = x.shape[1]
    DH = w2.shape[2]
    nblk = n_rows // block_m
    return pl.pallas_call(
        _linear_body,
        grid=(nblk, 2),
        in_specs=[
            pl.BlockSpec((block_m, K), lambda i, h: (i, 0)),
            pl.BlockSpec((1, K, DH), lambda i, h: (h, 0, 0)),
            pl.BlockSpec((1, 1, DH), lambda i, h: (h, 0, 0)),
        ],
        out_specs=pl.BlockSpec((block_m, DH), lambda i, h, _n=nblk:
                               (h * _n + i, 0)),
        out_shape=jax.ShapeDtypeStruct((2 * n_rows, DH), jnp.float32),
    )(x, w2, b2)


def _layer_body(a_ref, w_ref, b_ref, o_ref, *, dh):
    x = (jnp.dot(a_ref[0], w_ref[0, :dh, :],
                 preferred_element_type=jnp.float32)
         + jnp.dot(a_ref[1], w_ref[0, dh:, :],
                   preferred_element_type=jnp.float32))
    o_ref[...] = jnp.maximum(x + b_ref[0], 0.0)


def _layer_split(agg, w2, b2, n_rows, block_m):
    NCpart, _, DH = agg.shape
    nblk = n_rows // block_m
    return pl.pallas_call(
        functools.partial(_layer_body, dh=DH),
        grid=(nblk, 2),
        in_specs=[
            pl.BlockSpec((NCpart, block_m, DH), lambda i, h: (0, i, 0)),
            pl.BlockSpec((1, 2 * DH, DH), lambda i, h: (h, 0, 0)),
            pl.BlockSpec((1, 1, DH), lambda i, h: (h, 0, 0)),
        ],
        out_specs=pl.BlockSpec((block_m, DH), lambda i, h, _n=nblk:
                               (h * _n + i, 0)),
        out_shape=jax.ShapeDtypeStruct((2 * n_rows, DH), jnp.float32),
    )(agg, w2, b2)


def _final_body(a_ref, w_ref, b_ref, wc_ref, bc_ref, o_ref, *, dh, n_nodes):
    i = pl.program_id(0)
    x = (jnp.dot(a_ref[0], w_ref[:dh, :],
                 preferred_element_type=jnp.float32)
         + jnp.dot(a_ref[1], w_ref[dh:, :],
                   preferred_element_type=jnp.float32))
    hblk = jnp.maximum(x + b_ref[...], 0.0)
    part = jnp.sum(hblk * wc_ref[...]) / n_nodes
    prev = jnp.where(i == 0, bc_ref[0, 0], o_ref[0, 0])
    o_ref[0, 0] = prev + part


def _final(agg, w, b, wc_row, bc, n_rows, block_m):
    NCpart, _, DH = agg.shape
    D = w.shape[1]
    return pl.pallas_call(
        functools.partial(_final_body, dh=DH, n_nodes=n_rows),
        grid=(n_rows // block_m,),
        in_specs=[
            pl.BlockSpec((NCpart, block_m, DH), lambda i: (0, i, 0)),
            pl.BlockSpec((2 * DH, D), lambda i: (0, 0)),
            pl.BlockSpec((1, D), lambda i: (0, 0)),
            pl.BlockSpec((1, D), lambda i: (0, 0)),
            pl.BlockSpec(memory_space=pltpu.SMEM),
        ],
        out_specs=pl.BlockSpec(memory_space=pltpu.SMEM),
        out_shape=jax.ShapeDtypeStruct((1, 1), jnp.float32),
    )(agg, w, b, wc_row, bc)


def kernel(edge_index, feat, W_feat, b_feat, W_layers, b_layers, W_cls, b_cls):
    N = feat.shape[0]
    D_hid = W_feat.shape[1]
    DH = D_hid // 2
    L = W_layers.shape[0]
    E = edge_index.shape[1]

    info = plsc.get_sparse_core_info()
    NC, NS = info.num_cores, info.num_subcores
    NCHUNK = -(-(-(-E // (NS * _CH))) // (8 * _QF)) * (8 * _QF)
    E_pad = NCHUNK * _CH * NS
    src = edge_index[0].astype(jnp.int32)
    dst = edge_index[1].astype(jnp.int32)
    src_p = jnp.concatenate([src, jnp.zeros((E_pad - E,), jnp.int32)])
    dst_p = jnp.concatenate([dst, jnp.full((E_pad - E,), N, jnp.int32)])
    src_mat = src_p.reshape(NS, NCHUNK * _CH)
    dst_mat = dst_p.reshape(NS, NCHUNK * _CH)
    NPAD = -(-(N + 1) // (NS * 128)) * (NS * 128)
    zeros_tile = jnp.zeros((128, DH), jnp.float32)

    edge_agg = _make_edge_agg(N, DH, NC, NS, NCHUNK, NPAD)

    def _colsplit(w):
        return w.reshape(w.shape[0], 2, DH).transpose(1, 0, 2)

    def _bsplit(b):
        return b.reshape(2, 1, DH)

    block_m = 1000
    h2 = _linear_split(feat, _colsplit(W_feat), _bsplit(b_feat), N, block_m)
    for i in range(L - 1):
        agg = edge_agg(h2, src_mat, dst_mat, zeros_tile)
        h2 = _layer_split(agg, _colsplit(W_layers[i]), _bsplit(b_layers[i]),
                          N, block_m)
    agg = edge_agg(h2, src_mat, dst_mat, zeros_tile)
    p = _final(agg, W_layers[L - 1], b_layers[L - 1].reshape(1, D_hid),
               W_cls.reshape(1, D_hid), b_cls.reshape(1, 1), N, block_m)
    return p.reshape(1)


# speedup vs baseline: 9.3047x; 1.1630x over previous
import functools

import jax
import jax.numpy as jnp
from jax import lax
from jax.experimental import pallas as pl
from jax.experimental.pallas import tpu as pltpu
from jax.experimental.pallas import tpu_sc as plsc

_CH = 128
_QF = 5
_NB = 4


@functools.lru_cache(maxsize=None)
def _make_edge_agg(N, DH, NC, NS, NCHUNK, NPAD):
    NZ = NPAD // (NS * 128)
    ROWS_T = NPAD // NS
    QCH = NCHUNK // _QF
    HROWS = N // NS
    assert NCHUNK % _QF == 0 and QCH % 8 == 0 and QCH % _NB == 0
    mesh = plsc.VectorSubcoreMesh(core_axis_name="c", subcore_axis_name="s",
                                  num_cores=NC, num_subcores=NS)

    @functools.partial(
        pl.kernel,
        out_type=jax.ShapeDtypeStruct((NC, NPAD, DH), jnp.float32),
        mesh=mesh,
        compiler_params=pltpu.CompilerParams(use_tc_tiling_on_sc=False),
        scratch_types=[
            pltpu.VMEM((QCH * _CH,), jnp.int32),
            pltpu.VMEM((QCH * _CH,), jnp.int32),
            [pltpu.VMEM((_CH, DH), jnp.float32) for _ in range(_NB)],
            [pltpu.SemaphoreType.DMA for _ in range(_NB)],
            [pltpu.SemaphoreType.DMA for _ in range(_NB)],
            pltpu.VMEM_SHARED((N, DH), jnp.float32),
            pltpu.VMEM_SHARED((NPAD, DH), jnp.float32),
        ],
    )
    def edge_agg(h_hbm, src_hbm, dst_hbm, zeros_hbm, out_hbm,
                 src_v, dst_v, bufs, gsem, ssem, h_sh, agg_sh):
        c = lax.axis_index("c")
        s = lax.axis_index("s")

        def gather(j, b):
            pltpu.async_copy(h_sh.at[src_v.at[pl.ds(j * _CH, _CH)]],
                             bufs[b], gsem[b])

        def gather_wait(j, b):
            pltpu.make_async_copy(h_sh.at[src_v.at[pl.ds(j * _CH, _CH)]],
                                  bufs[b], gsem[b]).wait()

        def scatter(j, b):
            pltpu.async_copy(bufs[b],
                             agg_sh.at[dst_v.at[pl.ds(j * _CH, _CH)]],
                             ssem[b], add=True)

        def scatter_wait(j, b):
            pltpu.make_async_copy(bufs[b],
                                  agg_sh.at[dst_v.at[pl.ds(j * _CH, _CH)]],
                                  ssem[b]).wait()
        pltpu.sync_copy(h_hbm.at[pl.ds(c * N + s * HROWS, HROWS)],
                        h_sh.at[pl.ds(s * HROWS, HROWS)])
        pltpu.sync_copy(zeros_hbm, bufs[0].at[pl.ds(0, 128)])

        def zero_step(t, carry):
            pltpu.sync_copy(bufs[0].at[pl.ds(0, 128)],
                            agg_sh.at[pl.ds((s * NZ + t) * 128, 128)])
            return carry

        lax.fori_loop(0, NZ, zero_step, 0)
        plsc.subcore_barrier()

        NT = QCH // _NB

        def stage(q, carry):
            pltpu.sync_copy(src_hbm.at[s, pl.ds(q * QCH * _CH, QCH * _CH)],
                            src_v)
            pltpu.sync_copy(dst_hbm.at[s, pl.ds(q * QCH * _CH, QCH * _CH)],
                            dst_v)
            for b in range(_NB - 1):
                gather(b, b)

            def step(t, carry2):
                for u in range(_NB):
                    j = _NB * t + u
                    b = u
                    gather_wait(j, b)
                    scatter(j, b)
                    bn = (u + _NB - 1) % _NB
                    if u == 0:
                        @pl.when(t > 0)
                        def _():
                            scatter_wait(j - 1, bn)
                        gather(j + _NB - 1, bn)
                    else:
                        scatter_wait(j - 1, bn)

                        @pl.when(t < NT - 1)
                        def _():
                            gather(j + _NB - 1, bn)
                return carry2

            lax.fori_loop(0, NT, step, 0)
            scatter_wait(QCH - 1, _NB - 1)
            return carry

        lax.fori_loop(0, _QF, stage, 0)
        plsc.subcore_barrier()
        pltpu.sync_copy(agg_sh.at[pl.ds(s * ROWS_T, ROWS_T)],
                        out_hbm.at[c, pl.ds(s * ROWS_T, ROWS_T)])

    return edge_agg


def _linear_body(x_ref, w_ref, b_ref, o_ref):
    o_ref[...] = (jnp.dot(x_ref[...], w_ref[0],
                          preferred_element_type=jnp.float32) + b_ref[0])


def _linear_split(x, w2, b2, n_rows, block_m):
    K = x.shape[1]
    DH = w2.shape[2]
    nblk = n_rows // block_m
    return pl.pallas_call(
        _linear_body,
        grid=(nblk, 2),
        in_specs=[
            pl.BlockSpec((block_m, K), lambda i, h: (i, 0)),
            pl.BlockSpec((1, K, DH), lambda i, h: (h, 0, 0)),
            pl.BlockSpec((1, 1, DH), lambda i, h: (h, 0, 0)),
        ],
        out_specs=pl.BlockSpec((block_m, DH), lambda i, h, _n=nblk:
                               (h * _n + i, 0)),
        out_shape=jax.ShapeDtypeStruct((2 * n_rows, DH), jnp.float32),
    )(x, w2, b2)


def _layer_body(a_ref, w_ref, b_ref, o_ref, *, dh):
    x = (jnp.dot(a_ref[0], w_ref[0, :dh, :],
                 preferred_element_type=jnp.float32)
         + jnp.dot(a_ref[1], w_ref[0, dh:, :],
                   preferred_element_type=jnp.float32))
    o_ref[...] = jnp.maximum(x + b_ref[0], 0.0)


def _layer_split(agg, w2, b2, n_rows, block_m):
    NCpart, _, DH = agg.shape
    nblk = n_rows // block_m
    return pl.pallas_call(
        functools.partial(_layer_body, dh=DH),
        grid=(nblk, 2),
        in_specs=[
            pl.BlockSpec((NCpart, block_m, DH), lambda i, h: (0, i, 0)),
            pl.BlockSpec((1, 2 * DH, DH), lambda i, h: (h, 0, 0)),
            pl.BlockSpec((1, 1, DH), lambda i, h: (h, 0, 0)),
        ],
        out_specs=pl.BlockSpec((block_m, DH), lambda i, h, _n=nblk:
                               (h * _n + i, 0)),
        out_shape=jax.ShapeDtypeStruct((2 * n_rows, DH), jnp.float32),
    )(agg, w2, b2)


def _final_body(a_ref, w_ref, b_ref, wc_ref, bc_ref, o_ref, *, dh, n_nodes):
    i = pl.program_id(0)
    x = (jnp.dot(a_ref[0], w_ref[:dh, :],
                 preferred_element_type=jnp.float32)
         + jnp.dot(a_ref[1], w_ref[dh:, :],
                   preferred_element_type=jnp.float32))
    hblk = jnp.maximum(x + b_ref[...], 0.0)
    part = jnp.sum(hblk * wc_ref[...]) / n_nodes
    prev = jnp.where(i == 0, bc_ref[0, 0], o_ref[0, 0])
    o_ref[0, 0] = prev + part


def _final(agg, w, b, wc_row, bc, n_rows, block_m):
    NCpart, _, DH = agg.shape
    D = w.shape[1]
    return pl.pallas_call(
        functools.partial(_final_body, dh=DH, n_nodes=n_rows),
        grid=(n_rows // block_m,),
        in_specs=[
            pl.BlockSpec((NCpart, block_m, DH), lambda i: (0, i, 0)),
            pl.BlockSpec((2 * DH, D), lambda i: (0, 0)),
            pl.BlockSpec((1, D), lambda i: (0, 0)),
            pl.BlockSpec((1, D), lambda i: (0, 0)),
            pl.BlockSpec(memory_space=pltpu.SMEM),
        ],
        out_specs=pl.BlockSpec(memory_space=pltpu.SMEM),
        out_shape=jax.ShapeDtypeStruct((1, 1), jnp.float32),
    )(agg, w, b, wc_row, bc)


def kernel(edge_index, feat, W_feat, b_feat, W_layers, b_layers, W_cls, b_cls):
    N = feat.shape[0]
    D_hid = W_feat.shape[1]
    DH = D_hid // 2
    L = W_layers.shape[0]
    E = edge_index.shape[1]

    info = plsc.get_sparse_core_info()
    NC, NS = info.num_cores, info.num_subcores
    NCHUNK = -(-(-(-E // (NS * _CH))) // (8 * _QF)) * (8 * _QF)
    E_pad = NCHUNK * _CH * NS
    src = edge_index[0].astype(jnp.int32)
    dst = edge_index[1].astype(jnp.int32)
    src_p = jnp.concatenate([src, jnp.zeros((E_pad - E,), jnp.int32)])
    dst_p = jnp.concatenate([dst, jnp.full((E_pad - E,), N, jnp.int32)])
    src_mat = src_p.reshape(NS, NCHUNK * _CH)
    dst_mat = dst_p.reshape(NS, NCHUNK * _CH)
    NPAD = -(-(N + 1) // (NS * 128)) * (NS * 128)
    zeros_tile = jnp.zeros((128, DH), jnp.float32)

    edge_agg = _make_edge_agg(N, DH, NC, NS, NCHUNK, NPAD)

    def _colsplit(w):
        return w.reshape(w.shape[0], 2, DH).transpose(1, 0, 2)

    def _bsplit(b):
        return b.reshape(2, 1, DH)

    block_m = 1000
    h2 = _linear_split(feat, _colsplit(W_feat), _bsplit(b_feat), N, block_m)
    for i in range(L - 1):
        agg = edge_agg(h2, src_mat, dst_mat, zeros_tile)
        h2 = _layer_split(agg, _colsplit(W_layers[i]), _bsplit(b_layers[i]),
                          N, block_m)
    agg = edge_agg(h2, src_mat, dst_mat, zeros_tile)
    p = _final(agg, W_layers[L - 1], b_layers[L - 1].reshape(1, D_hid),
               W_cls.reshape(1, D_hid), b_cls.reshape(1, 1), N, block_m)
    return p.reshape(1)


# speedup vs baseline: 9.4216x; 1.0126x over previous
import functools

import jax
import jax.numpy as jnp
from jax import lax
from jax.experimental import pallas as pl
from jax.experimental.pallas import tpu as pltpu
from jax.experimental.pallas import tpu_sc as plsc

_CH = 128
_QF = 5
_NB = 4


@functools.lru_cache(maxsize=None)
def _make_edge_agg(N, DH, NC, NS, NCHUNK, NPAD):
    NZ = NPAD // (NS * 128)
    ROWS_T = NPAD // NS
    QCH = NCHUNK // _QF
    HROWS = N // NS
    assert NCHUNK % _QF == 0 and QCH % 8 == 0 and QCH % _NB == 0
    mesh = plsc.VectorSubcoreMesh(core_axis_name="c", subcore_axis_name="s",
                                  num_cores=NC, num_subcores=NS)

    @functools.partial(
        pl.kernel,
        out_type=jax.ShapeDtypeStruct((NC, NPAD, DH), jnp.float32),
        mesh=mesh,
        compiler_params=pltpu.CompilerParams(use_tc_tiling_on_sc=False),
        scratch_types=[
            pltpu.VMEM((QCH * _CH,), jnp.int32),
            pltpu.VMEM((QCH * _CH,), jnp.int32),
            [pltpu.VMEM((_CH, DH), jnp.float32) for _ in range(_NB)],
            [pltpu.SemaphoreType.DMA for _ in range(_NB)],
            [pltpu.SemaphoreType.DMA for _ in range(_NB)],
            pltpu.VMEM_SHARED((N, DH), jnp.float32),
            pltpu.VMEM_SHARED((NPAD, DH), jnp.float32),
        ],
    )
    def edge_agg(h_hbm, src_hbm, dst_hbm, zeros_hbm, out_hbm,
                 src_v, dst_v, bufs, gsem, ssem, h_sh, agg_sh):
        c = lax.axis_index("c")
        s = lax.axis_index("s")

        def gather(j, b):
            pltpu.async_copy(h_sh.at[src_v.at[pl.ds(j * _CH, _CH)]],
                             bufs[b], gsem[b])

        def gather_wait(j, b):
            pltpu.make_async_copy(h_sh.at[src_v.at[pl.ds(j * _CH, _CH)]],
                                  bufs[b], gsem[b]).wait()

        def scatter(j, b):
            pltpu.async_copy(bufs[b],
                             agg_sh.at[dst_v.at[pl.ds(j * _CH, _CH)]],
                             ssem[b], add=True)

        def scatter_wait(j, b):
            pltpu.make_async_copy(bufs[b],
                                  agg_sh.at[dst_v.at[pl.ds(j * _CH, _CH)]],
                                  ssem[b]).wait()
        pltpu.async_copy(h_hbm.at[pl.ds(c * N + s * HROWS, HROWS)],
                         h_sh.at[pl.ds(s * HROWS, HROWS)], ssem[0])
        pltpu.async_copy(src_hbm.at[s, pl.ds(0, QCH * _CH)], src_v, gsem[0])
        pltpu.async_copy(dst_hbm.at[s, pl.ds(0, QCH * _CH)], dst_v, gsem[1])
        pltpu.sync_copy(zeros_hbm, bufs[0].at[pl.ds(0, 128)])

        def zero_step(t, carry):
            pltpu.sync_copy(bufs[0].at[pl.ds(0, 128)],
                            agg_sh.at[pl.ds((s * NZ + t) * 128, 128)])
            return carry

        lax.fori_loop(0, NZ, zero_step, 0)
        pltpu.make_async_copy(h_hbm.at[pl.ds(c * N + s * HROWS, HROWS)],
                              h_sh.at[pl.ds(s * HROWS, HROWS)], ssem[0]).wait()
        pltpu.make_async_copy(src_hbm.at[s, pl.ds(0, QCH * _CH)], src_v,
                              gsem[0]).wait()
        pltpu.make_async_copy(dst_hbm.at[s, pl.ds(0, QCH * _CH)], dst_v,
                              gsem[1]).wait()
        plsc.subcore_barrier()

        NT = QCH // _NB

        def stage(q, carry):
            @pl.when(q > 0)
            def _():
                pltpu.sync_copy(src_hbm.at[s, pl.ds(q * QCH * _CH,
                                                    QCH * _CH)], src_v)
                pltpu.sync_copy(dst_hbm.at[s, pl.ds(q * QCH * _CH,
                                                    QCH * _CH)], dst_v)
            for b in range(_NB - 1):
                gather(b, b)

            def step(t, carry2):
                for u in range(_NB):
                    j = _NB * t + u
                    b = u
                    gather_wait(j, b)
                    scatter(j, b)
                    bn = (u + _NB - 1) % _NB
                    if u == 0:
                        @pl.when(t > 0)
                        def _():
                            scatter_wait(j - 1, bn)
                        gather(j + _NB - 1, bn)
                    else:
                        scatter_wait(j - 1, bn)

                        @pl.when(t < NT - 1)
                        def _():
                            gather(j + _NB - 1, bn)
                return carry2

            lax.fori_loop(0, NT, step, 0)
            scatter_wait(QCH - 1, _NB - 1)
            return carry

        lax.fori_loop(0, _QF, stage, 0)
        plsc.subcore_barrier()
        pltpu.sync_copy(agg_sh.at[pl.ds(s * ROWS_T, ROWS_T)],
                        out_hbm.at[c, pl.ds(s * ROWS_T, ROWS_T)])

    return edge_agg


def _linear_body(x_ref, w_ref, b_ref, o_ref):
    o_ref[...] = (jnp.dot(x_ref[...], w_ref[0],
                          preferred_element_type=jnp.float32) + b_ref[0])


def _linear_split(x, w2, b2, n_rows, block_m):
    K = x.shape[1]
    DH = w2.shape[2]
    nblk = n_rows // block_m
    return pl.pallas_call(
        _linear_body,
        grid=(nblk, 2),
        in_specs=[
            pl.BlockSpec((block_m, K), lambda i, h: (i, 0)),
            pl.BlockSpec((1, K, DH), lambda i, h: (h, 0, 0)),
            pl.BlockSpec((1, 1, DH), lambda i, h: (h, 0, 0)),
        ],
        out_specs=pl.BlockSpec((block_m, DH), lambda i, h, _n=nblk:
                               (h * _n + i, 0)),
        out_shape=jax.ShapeDtypeStruct((2 * n_rows, DH), jnp.float32),
    )(x, w2, b2)


def _layer_body(a_ref, w_ref, b_ref, o_ref, *, dh):
    x = (jnp.dot(a_ref[0], w_ref[0, :dh, :],
                 preferred_element_type=jnp.float32)
         + jnp.dot(a_ref[1], w_ref[0, dh:, :],
                   preferred_element_type=jnp.float32))
    o_ref[...] = jnp.maximum(x + b_ref[0], 0.0)


def _layer_split(agg, w2, b2, n_rows, block_m):
    NCpart, _, DH = agg.shape
    nblk = n_rows // block_m
    return pl.pallas_call(
        functools.partial(_layer_body, dh=DH),
        grid=(nblk, 2),
        in_specs=[
            pl.BlockSpec((NCpart, block_m, DH), lambda i, h: (0, i, 0)),
            pl.BlockSpec((1, 2 * DH, DH), lambda i, h: (h, 0, 0)),
            pl.BlockSpec((1, 1, DH), lambda i, h: (h, 0, 0)),
        ],
        out_specs=pl.BlockSpec((block_m, DH), lambda i, h, _n=nblk:
                               (h * _n + i, 0)),
        out_shape=jax.ShapeDtypeStruct((2 * n_rows, DH), jnp.float32),
    )(agg, w2, b2)


def _final_body(a_ref, w_ref, b_ref, wc_ref, bc_ref, o_ref, *, dh, n_nodes):
    i = pl.program_id(0)
    x = (jnp.dot(a_ref[0], w_ref[:dh, :],
                 preferred_element_type=jnp.float32)
         + jnp.dot(a_ref[1], w_ref[dh:, :],
                   preferred_element_type=jnp.float32))
    hblk = jnp.maximum(x + b_ref[...], 0.0)
    part = jnp.sum(hblk * wc_ref[...]) / n_nodes
    prev = jnp.where(i == 0, bc_ref[0, 0], o_ref[0, 0])
    o_ref[0, 0] = prev + part


def _final(agg, w, b, wc_row, bc, n_rows, block_m):
    NCpart, _, DH = agg.shape
    D = w.shape[1]
    return pl.pallas_call(
        functools.partial(_final_body, dh=DH, n_nodes=n_rows),
        grid=(n_rows // block_m,),
        in_specs=[
            pl.BlockSpec((NCpart, block_m, DH), lambda i: (0, i, 0)),
            pl.BlockSpec((2 * DH, D), lambda i: (0, 0)),
            pl.BlockSpec((1, D), lambda i: (0, 0)),
            pl.BlockSpec((1, D), lambda i: (0, 0)),
            pl.BlockSpec(memory_space=pltpu.SMEM),
        ],
        out_specs=pl.BlockSpec(memory_space=pltpu.SMEM),
        out_shape=jax.ShapeDtypeStruct((1, 1), jnp.float32),
    )(agg, w, b, wc_row, bc)


def kernel(edge_index, feat, W_feat, b_feat, W_layers, b_layers, W_cls, b_cls):
    N = feat.shape[0]
    D_hid = W_feat.shape[1]
    DH = D_hid // 2
    L = W_layers.shape[0]
    E = edge_index.shape[1]

    info = plsc.get_sparse_core_info()
    NC, NS = info.num_cores, info.num_subcores
    NCHUNK = -(-(-(-E // (NS * _CH))) // (8 * _QF)) * (8 * _QF)
    E_pad = NCHUNK * _CH * NS
    src = edge_index[0].astype(jnp.int32)
    dst = edge_index[1].astype(jnp.int32)
    src_p = jnp.concatenate([src, jnp.zeros((E_pad - E,), jnp.int32)])
    dst_p = jnp.concatenate([dst, jnp.full((E_pad - E,), N, jnp.int32)])
    src_mat = src_p.reshape(NS, NCHUNK * _CH)
    dst_mat = dst_p.reshape(NS, NCHUNK * _CH)
    NPAD = -(-(N + 1) // (NS * 128)) * (NS * 128)
    zeros_tile = jnp.zeros((128, DH), jnp.float32)

    edge_agg = _make_edge_agg(N, DH, NC, NS, NCHUNK, NPAD)

    def _colsplit(w):
        return w.reshape(w.shape[0], 2, DH).transpose(1, 0, 2)

    def _bsplit(b):
        return b.reshape(2, 1, DH)

    block_m = 1000
    h2 = _linear_split(feat, _colsplit(W_feat), _bsplit(b_feat), N, block_m)
    for i in range(L - 1):
        agg = edge_agg(h2, src_mat, dst_mat, zeros_tile)
        h2 = _layer_split(agg, _colsplit(W_layers[i]), _bsplit(b_layers[i]),
                          N, block_m)
    agg = edge_agg(h2, src_mat, dst_mat, zeros_tile)
    p = _final(agg, W_layers[L - 1], b_layers[L - 1].reshape(1, D_hid),
               W_cls.reshape(1, D_hid), b_cls.reshape(1, 1), N, block_m)
    return p.reshape(1)


# speedup vs baseline: 9.5585x; 1.0145x over previous
import functools

import jax
import jax.numpy as jnp
from jax import lax
from jax.experimental import pallas as pl
from jax.experimental.pallas import tpu as pltpu
from jax.experimental.pallas import tpu_sc as plsc

_CH = 128
_QF = 5
_NB = 4


@functools.lru_cache(maxsize=None)
def _make_edge_agg(N, DH, NC, NS, NCHUNK, NPAD):
    NZ = NPAD // (NS * 128)
    ROWS_T = NPAD // NS
    QCH = NCHUNK // _QF
    HROWS = N // NS
    assert NCHUNK % _QF == 0 and QCH % 8 == 0 and QCH % _NB == 0
    mesh = plsc.VectorSubcoreMesh(core_axis_name="c", subcore_axis_name="s",
                                  num_cores=NC, num_subcores=NS)

    @functools.partial(
        pl.kernel,
        out_type=jax.ShapeDtypeStruct((NC, NPAD, DH), jnp.float32),
        mesh=mesh,
        compiler_params=pltpu.CompilerParams(use_tc_tiling_on_sc=False),
        scratch_types=[
            [pltpu.VMEM((QCH * _CH,), jnp.int32) for _ in range(2)],
            [pltpu.VMEM((QCH * _CH,), jnp.int32) for _ in range(2)],
            [pltpu.SemaphoreType.DMA for _ in range(2)],
            [pltpu.VMEM((_CH, DH), jnp.float32) for _ in range(_NB)],
            [pltpu.SemaphoreType.DMA for _ in range(_NB)],
            [pltpu.SemaphoreType.DMA for _ in range(_NB)],
            pltpu.VMEM_SHARED((N, DH), jnp.float32),
            pltpu.VMEM_SHARED((NPAD, DH), jnp.float32),
        ],
    )
    def edge_agg(h_hbm, src_hbm, dst_hbm, zeros_hbm, out_hbm,
                 src_vs, dst_vs, isem, bufs, gsem, ssem, h_sh, agg_sh):
        c = lax.axis_index("c")
        s = lax.axis_index("s")

        def gather(j, b, sv):
            pltpu.async_copy(h_sh.at[sv.at[pl.ds(j * _CH, _CH)]],
                             bufs[b], gsem[b])

        def gather_wait(j, b, sv):
            pltpu.make_async_copy(h_sh.at[sv.at[pl.ds(j * _CH, _CH)]],
                                  bufs[b], gsem[b]).wait()

        def scatter(j, b, dv):
            pltpu.async_copy(bufs[b],
                             agg_sh.at[dv.at[pl.ds(j * _CH, _CH)]],
                             ssem[b], add=True)

        def scatter_wait(j, b, dv):
            pltpu.make_async_copy(bufs[b],
                                  agg_sh.at[dv.at[pl.ds(j * _CH, _CH)]],
                                  ssem[b]).wait()

        def stage_idx_start(q):
            pltpu.async_copy(src_hbm.at[s, pl.ds(q * QCH * _CH, QCH * _CH)],
                             src_vs[q % 2], isem[0])
            pltpu.async_copy(dst_hbm.at[s, pl.ds(q * QCH * _CH, QCH * _CH)],
                             dst_vs[q % 2], isem[1])

        def stage_idx_wait(q):
            pltpu.make_async_copy(
                src_hbm.at[s, pl.ds(q * QCH * _CH, QCH * _CH)],
                src_vs[q % 2], isem[0]).wait()
            pltpu.make_async_copy(
                dst_hbm.at[s, pl.ds(q * QCH * _CH, QCH * _CH)],
                dst_vs[q % 2], isem[1]).wait()
        pltpu.async_copy(h_hbm.at[pl.ds(c * N + s * HROWS, HROWS)],
                         h_sh.at[pl.ds(s * HROWS, HROWS)], ssem[0])
        stage_idx_start(0)
        pltpu.sync_copy(zeros_hbm, bufs[0].at[pl.ds(0, 128)])

        def zero_step(t, carry):
            pltpu.sync_copy(bufs[0].at[pl.ds(0, 128)],
                            agg_sh.at[pl.ds((s * NZ + t) * 128, 128)])
            return carry

        lax.fori_loop(0, NZ, zero_step, 0)
        pltpu.make_async_copy(h_hbm.at[pl.ds(c * N + s * HROWS, HROWS)],
                              h_sh.at[pl.ds(s * HROWS, HROWS)], ssem[0]).wait()
        plsc.subcore_barrier()

        NT = QCH // _NB

        for q in range(_QF):
            sv, dv = src_vs[q % 2], dst_vs[q % 2]
            stage_idx_wait(q)
            if q + 1 < _QF:
                stage_idx_start(q + 1)
            for b in range(_NB - 1):
                gather(b, b, sv)

            def step(t, carry2, sv=sv, dv=dv):
                for u in range(_NB):
                    j = _NB * t + u
                    b = u
                    gather_wait(j, b, sv)
                    scatter(j, b, dv)
                    bn = (u + _NB - 1) % _NB
                    if u == 0:
                        @pl.when(t > 0)
                        def _():
                            scatter_wait(j - 1, bn, dv)
                        gather(j + _NB - 1, bn, sv)
                    else:
                        scatter_wait(j - 1, bn, dv)

                        @pl.when(t < NT - 1)
                        def _():
                            gather(j + _NB - 1, bn, sv)
                return carry2

            lax.fori_loop(0, NT, step, 0)
            scatter_wait(QCH - 1, _NB - 1, dv)
        plsc.subcore_barrier()
        pltpu.sync_copy(agg_sh.at[pl.ds(s * ROWS_T, ROWS_T)],
                        out_hbm.at[c, pl.ds(s * ROWS_T, ROWS_T)])

    return edge_agg


def _linear_body(x_ref, w_ref, b_ref, o_ref):
    o_ref[...] = (jnp.dot(x_ref[...], w_ref[0],
                          preferred_element_type=jnp.float32) + b_ref[0])


def _linear_split(x, w2, b2, n_rows, block_m):
    K = x.shape[1]
    DH = w2.shape[2]
    nblk = n_rows // block_m
    return pl.pallas_call(
        _linear_body,
        grid=(nblk, 2),
        in_specs=[
            pl.BlockSpec((block_m, K), lambda i, h: (i, 0)),
            pl.BlockSpec((1, K, DH), lambda i, h: (h, 0, 0)),
            pl.BlockSpec((1, 1, DH), lambda i, h: (h, 0, 0)),
        ],
        out_specs=pl.BlockSpec((block_m, DH), lambda i, h, _n=nblk:
                               (h * _n + i, 0)),
        out_shape=jax.ShapeDtypeStruct((2 * n_rows, DH), jnp.float32),
    )(x, w2, b2)


def _layer_body(a_ref, w_ref, b_ref, o_ref, *, dh):
    x = (jnp.dot(a_ref[0], w_ref[0, :dh, :],
                 preferred_element_type=jnp.float32)
         + jnp.dot(a_ref[1], w_ref[0, dh:, :],
                   preferred_element_type=jnp.float32))
    o_ref[...] = jnp.maximum(x + b_ref[0], 0.0)


def _layer_split(agg, w2, b2, n_rows, block_m):
    NCpart, _, DH = agg.shape
    nblk = n_rows // block_m
    return pl.pallas_call(
        functools.partial(_layer_body, dh=DH),
        grid=(nblk, 2),
        in_specs=[
            pl.BlockSpec((NCpart, block_m, DH), lambda i, h: (0, i, 0)),
            pl.BlockSpec((1, 2 * DH, DH), lambda i, h: (h, 0, 0)),
            pl.BlockSpec((1, 1, DH), lambda i, h: (h, 0, 0)),
        ],
        out_specs=pl.BlockSpec((block_m, DH), lambda i, h, _n=nblk:
                               (h * _n + i, 0)),
        out_shape=jax.ShapeDtypeStruct((2 * n_rows, DH), jnp.float32),
    )(agg, w2, b2)


def _final_body(a_ref, w_ref, b_ref, wc_ref, bc_ref, o_ref, *, dh, n_nodes):
    i = pl.program_id(0)
    x = (jnp.dot(a_ref[0], w_ref[:dh, :],
                 preferred_element_type=jnp.float32)
         + jnp.dot(a_ref[1], w_ref[dh:, :],
                   preferred_element_type=jnp.float32))
    hblk = jnp.maximum(x + b_ref[...], 0.0)
    part = jnp.sum(hblk * wc_ref[...]) / n_nodes
    prev = jnp.where(i == 0, bc_ref[0, 0], o_ref[0, 0])
    o_ref[0, 0] = prev + part


def _final(agg, w, b, wc_row, bc, n_rows, block_m):
    NCpart, _, DH = agg.shape
    D = w.shape[1]
    return pl.pallas_call(
        functools.partial(_final_body, dh=DH, n_nodes=n_rows),
        grid=(n_rows // block_m,),
        in_specs=[
            pl.BlockSpec((NCpart, block_m, DH), lambda i: (0, i, 0)),
            pl.BlockSpec((2 * DH, D), lambda i: (0, 0)),
            pl.BlockSpec((1, D), lambda i: (0, 0)),
            pl.BlockSpec((1, D), lambda i: (0, 0)),
            pl.BlockSpec(memory_space=pltpu.SMEM),
        ],
        out_specs=pl.BlockSpec(memory_space=pltpu.SMEM),
        out_shape=jax.ShapeDtypeStruct((1, 1), jnp.float32),
    )(agg, w, b, wc_row, bc)


def kernel(edge_index, feat, W_feat, b_feat, W_layers, b_layers, W_cls, b_cls):
    N = feat.shape[0]
    D_hid = W_feat.shape[1]
    DH = D_hid // 2
    L = W_layers.shape[0]
    E = edge_index.shape[1]

    info = plsc.get_sparse_core_info()
    NC, NS = info.num_cores, info.num_subcores
    NCHUNK = -(-(-(-E // (NS * _CH))) // (8 * _QF)) * (8 * _QF)
    E_pad = NCHUNK * _CH * NS
    src = edge_index[0].astype(jnp.int32)
    dst = edge_index[1].astype(jnp.int32)
    src_p = jnp.concatenate([src, jnp.zeros((E_pad - E,), jnp.int32)])
    dst_p = jnp.concatenate([dst, jnp.full((E_pad - E,), N, jnp.int32)])
    src_mat = src_p.reshape(NS, NCHUNK * _CH)
    dst_mat = dst_p.reshape(NS, NCHUNK * _CH)
    NPAD = -(-(N + 1) // (NS * 128)) * (NS * 128)
    zeros_tile = jnp.zeros((128, DH), jnp.float32)

    edge_agg = _make_edge_agg(N, DH, NC, NS, NCHUNK, NPAD)

    def _colsplit(w):
        return w.reshape(w.shape[0], 2, DH).transpose(1, 0, 2)

    def _bsplit(b):
        return b.reshape(2, 1, DH)

    block_m = 1000
    h2 = _linear_split(feat, _colsplit(W_feat), _bsplit(b_feat), N, block_m)
    for i in range(L - 1):
        agg = edge_agg(h2, src_mat, dst_mat, zeros_tile)
        h2 = _layer_split(agg, _colsplit(W_layers[i]), _bsplit(b_layers[i]),
                          N, block_m)
    agg = edge_agg(h2, src_mat, dst_mat, zeros_tile)
    p = _final(agg, W_layers[L - 1], b_layers[L - 1].reshape(1, D_hid),
               W_cls.reshape(1, D_hid), b_cls.reshape(1, 1), N, block_m)
    return p.reshape(1)


# speedup vs baseline: 9.5783x; 1.0021x over previous
import functools

import jax
import jax.numpy as jnp
from jax import lax
from jax.experimental import pallas as pl
from jax.experimental.pallas import tpu as pltpu
from jax.experimental.pallas import tpu_sc as plsc

_CH = 128
_QF = 5
_NB = 4


@functools.lru_cache(maxsize=None)
def _make_edge_agg(N, DH, NC, NS, NCHUNK, NPAD):
    NZ = NPAD // (NS * 128)
    ROWS_T = NPAD // NS
    QCH = NCHUNK // _QF
    HROWS = N // NS
    assert NCHUNK % _QF == 0 and QCH % 8 == 0 and QCH % _NB == 0
    mesh = plsc.VectorSubcoreMesh(core_axis_name="c", subcore_axis_name="s",
                                  num_cores=NC, num_subcores=NS)

    @functools.partial(
        pl.kernel,
        out_type=jax.ShapeDtypeStruct((NC, NPAD, DH), jnp.float32),
        mesh=mesh,
        compiler_params=pltpu.CompilerParams(use_tc_tiling_on_sc=False),
        scratch_types=[
            [pltpu.VMEM((QCH * _CH,), jnp.int32) for _ in range(2)],
            [pltpu.VMEM((QCH * _CH,), jnp.int32) for _ in range(2)],
            [pltpu.SemaphoreType.DMA for _ in range(2)],
            [pltpu.VMEM((_CH, DH), jnp.float32) for _ in range(_NB)],
            [pltpu.SemaphoreType.DMA for _ in range(_NB)],
            [pltpu.SemaphoreType.DMA for _ in range(_NB)],
            pltpu.VMEM_SHARED((N, DH), jnp.float32),
            pltpu.VMEM_SHARED((NPAD, DH), jnp.float32),
        ],
    )
    def edge_agg(h_hbm, src_hbm, dst_hbm, zeros_hbm, out_hbm,
                 src_vs, dst_vs, isem, bufs, gsem, ssem, h_sh, agg_sh):
        c = lax.axis_index("c")
        s = lax.axis_index("s")

        def gather(j, b, sv):
            pltpu.async_copy(h_sh.at[sv.at[pl.ds(j * _CH, _CH)]],
                             bufs[b], gsem[b])

        def gather_wait(j, b, sv):
            pltpu.make_async_copy(h_sh.at[sv.at[pl.ds(j * _CH, _CH)]],
                                  bufs[b], gsem[b]).wait()

        def scatter(j, b, dv):
            pltpu.async_copy(bufs[b],
                             agg_sh.at[dv.at[pl.ds(j * _CH, _CH)]],
                             ssem[b], add=True)

        def scatter_wait(j, b, dv):
            pltpu.make_async_copy(bufs[b],
                                  agg_sh.at[dv.at[pl.ds(j * _CH, _CH)]],
                                  ssem[b]).wait()

        def stage_idx_start(q):
            pltpu.async_copy(src_hbm.at[s, pl.ds(q * QCH * _CH, QCH * _CH)],
                             src_vs[q % 2], isem[0])
            pltpu.async_copy(dst_hbm.at[s, pl.ds(q * QCH * _CH, QCH * _CH)],
                             dst_vs[q % 2], isem[1])

        def stage_idx_wait(q):
            pltpu.make_async_copy(
                src_hbm.at[s, pl.ds(q * QCH * _CH, QCH * _CH)],
                src_vs[q % 2], isem[0]).wait()
            pltpu.make_async_copy(
                dst_hbm.at[s, pl.ds(q * QCH * _CH, QCH * _CH)],
                dst_vs[q % 2], isem[1]).wait()
        pltpu.async_copy(h_hbm.at[pl.ds(c * N + s * HROWS, HROWS)],
                         h_sh.at[pl.ds(s * HROWS, HROWS)], ssem[0])
        stage_idx_start(0)
        pltpu.sync_copy(zeros_hbm, bufs[0].at[pl.ds(0, 128)])

        for t in range(NZ):
            pltpu.async_copy(bufs[0].at[pl.ds(0, 128)],
                             agg_sh.at[pl.ds((s * NZ + t) * 128, 128)],
                             ssem[1])
        for t in range(NZ):
            pltpu.make_async_copy(bufs[0].at[pl.ds(0, 128)],
                                  agg_sh.at[pl.ds((s * NZ + t) * 128, 128)],
                                  ssem[1]).wait()
        pltpu.make_async_copy(h_hbm.at[pl.ds(c * N + s * HROWS, HROWS)],
                              h_sh.at[pl.ds(s * HROWS, HROWS)], ssem[0]).wait()
        plsc.subcore_barrier()

        NT = QCH // _NB

        for q in range(_QF):
            sv, dv = src_vs[q % 2], dst_vs[q % 2]
            stage_idx_wait(q)
            if q + 1 < _QF:
                stage_idx_start(q + 1)
            for b in range(_NB - 1):
                gather(b, b, sv)

            def step(t, carry2, sv=sv, dv=dv):
                for u in range(_NB):
                    j = _NB * t + u
                    b = u
                    gather_wait(j, b, sv)
                    scatter(j, b, dv)
                    bn = (u + _NB - 1) % _NB
                    if u == 0:
                        @pl.when(t > 0)
                        def _():
                            scatter_wait(j - 1, bn, dv)
                        gather(j + _NB - 1, bn, sv)
                    else:
                        scatter_wait(j - 1, bn, dv)

                        @pl.when(t < NT - 1)
                        def _():
                            gather(j + _NB - 1, bn, sv)
                return carry2

            lax.fori_loop(0, NT, step, 0)
            scatter_wait(QCH - 1, _NB - 1, dv)
        plsc.subcore_barrier()
        pltpu.sync_copy(agg_sh.at[pl.ds(s * ROWS_T, ROWS_T)],
                        out_hbm.at[c, pl.ds(s * ROWS_T, ROWS_T)])

    return edge_agg


def _linear_body(x_ref, w_ref, b_ref, o_ref):
    o_ref[...] = (jnp.dot(x_ref[...], w_ref[0],
                          preferred_element_type=jnp.float32) + b_ref[0])


def _linear_split(x, w2, b2, n_rows, block_m):
    K = x.shape[1]
    DH = w2.shape[2]
    nblk = n_rows // block_m
    return pl.pallas_call(
        _linear_body,
        grid=(nblk, 2),
        in_specs=[
            pl.BlockSpec((block_m, K), lambda i, h: (i, 0)),
            pl.BlockSpec((1, K, DH), lambda i, h: (h, 0, 0)),
            pl.BlockSpec((1, 1, DH), lambda i, h: (h, 0, 0)),
        ],
        out_specs=pl.BlockSpec((block_m, DH), lambda i, h, _n=nblk:
                               (h * _n + i, 0)),
        out_shape=jax.ShapeDtypeStruct((2 * n_rows, DH), jnp.float32),
    )(x, w2, b2)


def _layer_body(a_ref, w_ref, b_ref, o_ref, *, dh):
    x = (jnp.dot(a_ref[0], w_ref[0, :dh, :],
                 preferred_element_type=jnp.float32)
         + jnp.dot(a_ref[1], w_ref[0, dh:, :],
                   preferred_element_type=jnp.float32))
    o_ref[...] = jnp.maximum(x + b_ref[0], 0.0)


def _layer_split(agg, w2, b2, n_rows, block_m):
    NCpart, _, DH = agg.shape
    nblk = n_rows // block_m
    return pl.pallas_call(
        functools.partial(_layer_body, dh=DH),
        grid=(nblk, 2),
        in_specs=[
            pl.BlockSpec((NCpart, block_m, DH), lambda i, h: (0, i, 0)),
            pl.BlockSpec((1, 2 * DH, DH), lambda i, h: (h, 0, 0)),
            pl.BlockSpec((1, 1, DH), lambda i, h: (h, 0, 0)),
        ],
        out_specs=pl.BlockSpec((block_m, DH), lambda i, h, _n=nblk:
                               (h * _n + i, 0)),
        out_shape=jax.ShapeDtypeStruct((2 * n_rows, DH), jnp.float32),
    )(agg, w2, b2)


def _final_body(a_ref, w_ref, b_ref, wc_ref, bc_ref, o_ref, *, dh, n_nodes):
    i = pl.program_id(0)
    x = (jnp.dot(a_ref[0], w_ref[:dh, :],
                 preferred_element_type=jnp.float32)
         + jnp.dot(a_ref[1], w_ref[dh:, :],
                   preferred_element_type=jnp.float32))
    hblk = jnp.maximum(x + b_ref[...], 0.0)
    part = jnp.sum(hblk * wc_ref[...]) / n_nodes
    prev = jnp.where(i == 0, bc_ref[0, 0], o_ref[0, 0])
    o_ref[0, 0] = prev + part


def _final(agg, w, b, wc_row, bc, n_rows, block_m):
    NCpart, _, DH = agg.shape
    D = w.shape[1]
    return pl.pallas_call(
        functools.partial(_final_body, dh=DH, n_nodes=n_rows),
        grid=(n_rows // block_m,),
        in_specs=[
            pl.BlockSpec((NCpart, block_m, DH), lambda i: (0, i, 0)),
            pl.BlockSpec((2 * DH, D), lambda i: (0, 0)),
            pl.BlockSpec((1, D), lambda i: (0, 0)),
            pl.BlockSpec((1, D), lambda i: (0, 0)),
            pl.BlockSpec(memory_space=pltpu.SMEM),
        ],
        out_specs=pl.BlockSpec(memory_space=pltpu.SMEM),
        out_shape=jax.ShapeDtypeStruct((1, 1), jnp.float32),
    )(agg, w, b, wc_row, bc)


def kernel(edge_index, feat, W_feat, b_feat, W_layers, b_layers, W_cls, b_cls):
    N = feat.shape[0]
    D_hid = W_feat.shape[1]
    DH = D_hid // 2
    L = W_layers.shape[0]
    E = edge_index.shape[1]

    info = plsc.get_sparse_core_info()
    NC, NS = info.num_cores, info.num_subcores
    NCHUNK = -(-(-(-E // (NS * _CH))) // (8 * _QF)) * (8 * _QF)
    E_pad = NCHUNK * _CH * NS
    src = edge_index[0].astype(jnp.int32)
    dst = edge_index[1].astype(jnp.int32)
    src_p = jnp.concatenate([src, jnp.zeros((E_pad - E,), jnp.int32)])
    dst_p = jnp.concatenate([dst, jnp.full((E_pad - E,), N, jnp.int32)])
    src_mat = src_p.reshape(NS, NCHUNK * _CH)
    dst_mat = dst_p.reshape(NS, NCHUNK * _CH)
    NPAD = -(-(N + 1) // (NS * 128)) * (NS * 128)
    zeros_tile = jnp.zeros((128, DH), jnp.float32)

    edge_agg = _make_edge_agg(N, DH, NC, NS, NCHUNK, NPAD)

    def _colsplit(w):
        return w.reshape(w.shape[0], 2, DH).transpose(1, 0, 2)

    def _bsplit(b):
        return b.reshape(2, 1, DH)

    block_m = 1000
    h2 = _linear_split(feat, _colsplit(W_feat), _bsplit(b_feat), N, block_m)
    for i in range(L - 1):
        agg = edge_agg(h2, src_mat, dst_mat, zeros_tile)
        h2 = _layer_split(agg, _colsplit(W_layers[i]), _bsplit(b_layers[i]),
                          N, block_m)
    agg = edge_agg(h2, src_mat, dst_mat, zeros_tile)
    p = _final(agg, W_layers[L - 1], b_layers[L - 1].reshape(1, D_hid),
               W_cls.reshape(1, D_hid), b_cls.reshape(1, 1), N, block_m)
    return p.reshape(1)
